# baseline (speedup 1.0000x reference)
import jax
import jax.numpy as jnp
from jax.experimental import pallas as pl

INPUT_DIM = 1024
OUTPUT_DIM = 1024
NUM_EXPERTS = 8
TOPK = 2
TOKENS = 8192

BT = 1024


def _moe_body(x_ref, wg_ref, bg_ref, we_ref, be_ref, o_ref):
    x = x_ref[...]
    logits = jnp.dot(x, wg_ref[...], preferred_element_type=jnp.float32)
    logits = logits + bg_ref[...]
    probs = jax.nn.softmax(logits, axis=-1)
    rank = jnp.zeros(probs.shape, dtype=jnp.int32)
    idx = jax.lax.broadcasted_iota(jnp.int32, probs.shape, 1)
    for j in range(NUM_EXPERTS):
        pj = probs[:, j:j + 1]
        beat = (pj > probs) | ((pj == probs) & (j < idx))
        rank = rank + beat.astype(jnp.int32)
    w = jnp.where(rank < TOPK, probs, 0.0)
    xb = x.astype(jnp.bfloat16)
    acc = jnp.dot(w, be_ref[...], preferred_element_type=jnp.float32)
    for e in range(NUM_EXPERTS):
        we = we_ref[:, e * OUTPUT_DIM:(e + 1) * OUTPUT_DIM]
        y = jnp.dot(xb, we, preferred_element_type=jnp.float32)
        acc = acc + w[:, e:e + 1] * y
    o_ref[...] = acc


@jax.jit
def kernel(x, W_experts, b_experts, W_gate, b_gate):
    bg = b_gate.reshape(1, NUM_EXPERTS)
    be = b_experts.reshape(NUM_EXPERTS, OUTPUT_DIM)
    we = W_experts.astype(jnp.bfloat16)
    grid = (TOKENS // BT,)
    return pl.pallas_call(
        _moe_body,
        grid=grid,
        in_specs=[
            pl.BlockSpec((BT, INPUT_DIM), lambda t: (t, 0)),
            pl.BlockSpec((INPUT_DIM, NUM_EXPERTS), lambda t: (0, 0)),
            pl.BlockSpec((1, NUM_EXPERTS), lambda t: (0, 0)),
            pl.BlockSpec((INPUT_DIM, NUM_EXPERTS * OUTPUT_DIM), lambda t: (0, 0)),
            pl.BlockSpec((NUM_EXPERTS, OUTPUT_DIM), lambda t: (0, 0)),
        ],
        out_specs=pl.BlockSpec((BT, OUTPUT_DIM), lambda t: (t, 0)),
        out_shape=jax.ShapeDtypeStruct((TOKENS, OUTPUT_DIM), jnp.float32),
    )(x, W_gate, bg, we, be)


# speedup vs baseline: 1.0270x; 1.0270x over previous
import jax
import jax.numpy as jnp
from jax.experimental import pallas as pl
from jax.experimental.pallas import tpu as pltpu

INPUT_DIM = 1024
OUTPUT_DIM = 1024
NUM_EXPERTS = 8
TOPK = 2
TOKENS = 8192

BT = 256


def _moe_body(x_ref, wg_ref, bg_ref, we_ref, be_ref, o_ref, web_ref):
    t = pl.program_id(0)

    @pl.when(t == 0)
    def _cast_w():
        for e in range(NUM_EXPERTS):
            sl = slice(e * OUTPUT_DIM, (e + 1) * OUTPUT_DIM)
            web_ref[:, sl] = we_ref[:, sl].astype(jnp.bfloat16)

    x = x_ref[...]
    logits = jnp.dot(x, wg_ref[...], preferred_element_type=jnp.float32)
    logits = logits + bg_ref[...]
    probs = jax.nn.softmax(logits, axis=-1)
    rank = jnp.zeros(probs.shape, dtype=jnp.int32)
    idx = jax.lax.broadcasted_iota(jnp.int32, probs.shape, 1)
    for j in range(NUM_EXPERTS):
        pj = probs[:, j:j + 1]
        beat = (pj > probs) | ((pj == probs) & (j < idx))
        rank = rank + beat.astype(jnp.int32)
    w = jnp.where(rank < TOPK, probs, 0.0)
    xb = x.astype(jnp.bfloat16)
    acc = jnp.dot(w, be_ref[...], preferred_element_type=jnp.float32)
    for e in range(NUM_EXPERTS):
        we = web_ref[:, e * OUTPUT_DIM:(e + 1) * OUTPUT_DIM]
        y = jnp.dot(xb, we, preferred_element_type=jnp.float32)
        acc = acc + w[:, e:e + 1] * y
    o_ref[...] = acc


@jax.jit
def kernel(x, W_experts, b_experts, W_gate, b_gate):
    bg = b_gate.reshape(1, NUM_EXPERTS)
    be = b_experts.reshape(NUM_EXPERTS, OUTPUT_DIM)
    grid = (TOKENS // BT,)
    return pl.pallas_call(
        _moe_body,
        grid=grid,
        in_specs=[
            pl.BlockSpec((BT, INPUT_DIM), lambda t: (t, 0)),
            pl.BlockSpec((INPUT_DIM, NUM_EXPERTS), lambda t: (0, 0)),
            pl.BlockSpec((1, NUM_EXPERTS), lambda t: (0, 0)),
            pl.BlockSpec((INPUT_DIM, NUM_EXPERTS * OUTPUT_DIM), lambda t: (0, 0)),
            pl.BlockSpec((NUM_EXPERTS, OUTPUT_DIM), lambda t: (0, 0)),
        ],
        out_specs=pl.BlockSpec((BT, OUTPUT_DIM), lambda t: (t, 0)),
        out_shape=jax.ShapeDtypeStruct((TOKENS, OUTPUT_DIM), jnp.float32),
        scratch_shapes=[pltpu.VMEM((INPUT_DIM, NUM_EXPERTS * OUTPUT_DIM), jnp.bfloat16)],
    )(x, W_gate, bg, W_experts, be)


# speedup vs baseline: 1.0348x; 1.0076x over previous
import jax
import jax.numpy as jnp
from jax.experimental import pallas as pl
from jax.experimental.pallas import tpu as pltpu

INPUT_DIM = 1024
OUTPUT_DIM = 1024
NUM_EXPERTS = 8
TOPK = 2
TOKENS = 8192

BT = 512


def _moe_body(x_ref, wg_ref, bg_ref, we_ref, be_ref, o_ref, web_ref):
    t = pl.program_id(0)

    @pl.when(t == 0)
    def _cast_w():
        for e in range(NUM_EXPERTS):
            sl = slice(e * OUTPUT_DIM, (e + 1) * OUTPUT_DIM)
            web_ref[:, sl] = we_ref[:, sl].astype(jnp.bfloat16)

    x = x_ref[...]
    logits = jnp.dot(x, wg_ref[...], preferred_element_type=jnp.float32)
    logits = logits + bg_ref[...]
    probs = jax.nn.softmax(logits, axis=-1)
    rank = jnp.zeros(probs.shape, dtype=jnp.int32)
    idx = jax.lax.broadcasted_iota(jnp.int32, probs.shape, 1)
    for j in range(NUM_EXPERTS):
        pj = probs[:, j:j + 1]
        beat = (pj > probs) | ((pj == probs) & (j < idx))
        rank = rank + beat.astype(jnp.int32)
    w = jnp.where(rank < TOPK, probs, 0.0)
    xb = x.astype(jnp.bfloat16)
    acc = jnp.dot(w, be_ref[...], preferred_element_type=jnp.float32)
    for e in range(NUM_EXPERTS):
        we = web_ref[:, e * OUTPUT_DIM:(e + 1) * OUTPUT_DIM]
        y = jnp.dot(xb, we, preferred_element_type=jnp.float32)
        acc = acc + w[:, e:e + 1] * y
    o_ref[...] = acc


@jax.jit
def kernel(x, W_experts, b_experts, W_gate, b_gate):
    bg = b_gate.reshape(1, NUM_EXPERTS)
    be = b_experts.reshape(NUM_EXPERTS, OUTPUT_DIM)
    grid = (TOKENS // BT,)
    return pl.pallas_call(
        _moe_body,
        grid=grid,
        in_specs=[
            pl.BlockSpec((BT, INPUT_DIM), lambda t: (t, 0)),
            pl.BlockSpec((INPUT_DIM, NUM_EXPERTS), lambda t: (0, 0)),
            pl.BlockSpec((1, NUM_EXPERTS), lambda t: (0, 0)),
            pl.BlockSpec((INPUT_DIM, NUM_EXPERTS * OUTPUT_DIM), lambda t: (0, 0)),
            pl.BlockSpec((NUM_EXPERTS, OUTPUT_DIM), lambda t: (0, 0)),
        ],
        out_specs=pl.BlockSpec((BT, OUTPUT_DIM), lambda t: (t, 0)),
        out_shape=jax.ShapeDtypeStruct((TOKENS, OUTPUT_DIM), jnp.float32),
        scratch_shapes=[pltpu.VMEM((INPUT_DIM, NUM_EXPERTS * OUTPUT_DIM), jnp.bfloat16)],
        compiler_params=pltpu.CompilerParams(vmem_limit_bytes=100 * 1024 * 1024),
    )(x, W_gate, bg, W_experts, be)


# speedup vs baseline: 1.0424x; 1.0074x over previous
import jax
import jax.numpy as jnp
from jax.experimental import pallas as pl
from jax.experimental.pallas import tpu as pltpu

INPUT_DIM = 1024
OUTPUT_DIM = 1024
NUM_EXPERTS = 8
TOPK = 2
TOKENS = 8192

BT = 512


def _moe_body(x_ref, wg_ref, bg_ref, we_ref, be_ref, o_ref):
    x = x_ref[...]
    logits = jnp.dot(x, wg_ref[...], preferred_element_type=jnp.float32)
    logits = logits + bg_ref[...]
    probs = jax.nn.softmax(logits, axis=-1)
    rank = jnp.zeros(probs.shape, dtype=jnp.int32)
    idx = jax.lax.broadcasted_iota(jnp.int32, probs.shape, 1)
    for j in range(NUM_EXPERTS):
        pj = probs[:, j:j + 1]
        beat = (pj > probs) | ((pj == probs) & (j < idx))
        rank = rank + beat.astype(jnp.int32)
    w = jnp.where(rank < TOPK, probs, 0.0)
    xb = x.astype(jnp.bfloat16)
    acc = jnp.dot(w, be_ref[...], preferred_element_type=jnp.float32)
    for e in range(NUM_EXPERTS):
        we = we_ref[:, e * OUTPUT_DIM:(e + 1) * OUTPUT_DIM].astype(jnp.bfloat16)
        y = jnp.dot(xb, we, preferred_element_type=jnp.float32)
        acc = acc + w[:, e:e + 1] * y
    o_ref[...] = acc


@jax.jit
def kernel(x, W_experts, b_experts, W_gate, b_gate):
    bg = b_gate.reshape(1, NUM_EXPERTS)
    be = b_experts.reshape(NUM_EXPERTS, OUTPUT_DIM)
    grid = (TOKENS // BT,)
    return pl.pallas_call(
        _moe_body,
        grid=grid,
        in_specs=[
            pl.BlockSpec((BT, INPUT_DIM), lambda t: (t, 0)),
            pl.BlockSpec((INPUT_DIM, NUM_EXPERTS), lambda t: (0, 0)),
            pl.BlockSpec((1, NUM_EXPERTS), lambda t: (0, 0)),
            pl.BlockSpec((INPUT_DIM, NUM_EXPERTS * OUTPUT_DIM), lambda t: (0, 0)),
            pl.BlockSpec((NUM_EXPERTS, OUTPUT_DIM), lambda t: (0, 0)),
        ],
        out_specs=pl.BlockSpec((BT, OUTPUT_DIM), lambda t: (t, 0)),
        out_shape=jax.ShapeDtypeStruct((TOKENS, OUTPUT_DIM), jnp.float32),
    )(x, W_gate, bg, W_experts, be)


# speedup vs baseline: 1.1099x; 1.0647x over previous
import jax
import jax.numpy as jnp
from jax.experimental import pallas as pl
from jax.experimental.pallas import tpu as pltpu

INPUT_DIM = 1024
OUTPUT_DIM = 1024
NUM_EXPERTS = 8
TOPK = 2
TOKENS = 8192

BT = 512


def _moe_body(x_ref, wg_ref, bg_ref, we_ref, be_ref, o_ref):
    x = x_ref[...]
    logits = jnp.dot(x, wg_ref[...], preferred_element_type=jnp.float32)
    logits = logits + bg_ref[...]
    probs = jax.nn.softmax(logits, axis=-1)
    rank = jnp.zeros(probs.shape, dtype=jnp.int32)
    idx = jax.lax.broadcasted_iota(jnp.int32, probs.shape, 1)
    for j in range(NUM_EXPERTS):
        pj = probs[:, j:j + 1]
        beat = (pj > probs) | ((pj == probs) & (j < idx))
        rank = rank + beat.astype(jnp.int32)
    w = jnp.where(rank < TOPK, probs, 0.0)
    xb = x.astype(jnp.bfloat16)
    acc = jnp.zeros((x.shape[0], OUTPUT_DIM), dtype=jnp.float32)
    for e in range(NUM_EXPERTS):
        we = we_ref[:, e * OUTPUT_DIM:(e + 1) * OUTPUT_DIM].astype(jnp.bfloat16)
        y = jnp.dot(xb, we, preferred_element_type=jnp.float32)
        y = y + be_ref[0, e * OUTPUT_DIM:(e + 1) * OUTPUT_DIM][None, :]
        acc = acc + w[:, e:e + 1] * y
    o_ref[...] = acc


@jax.jit
def kernel(x, W_experts, b_experts, W_gate, b_gate):
    bg = b_gate.reshape(1, NUM_EXPERTS)
    be = b_experts.reshape(1, NUM_EXPERTS * OUTPUT_DIM)
    grid = (TOKENS // BT,)
    return pl.pallas_call(
        _moe_body,
        grid=grid,
        in_specs=[
            pl.BlockSpec((BT, INPUT_DIM), lambda t: (t, 0)),
            pl.BlockSpec((INPUT_DIM, NUM_EXPERTS), lambda t: (0, 0)),
            pl.BlockSpec((1, NUM_EXPERTS), lambda t: (0, 0)),
            pl.BlockSpec((INPUT_DIM, NUM_EXPERTS * OUTPUT_DIM), lambda t: (0, 0)),
            pl.BlockSpec((1, NUM_EXPERTS * OUTPUT_DIM), lambda t: (0, 0)),
        ],
        out_specs=pl.BlockSpec((BT, OUTPUT_DIM), lambda t: (t, 0)),
        out_shape=jax.ShapeDtypeStruct((TOKENS, OUTPUT_DIM), jnp.float32),
    )(x, W_gate, bg, W_experts, be)
